# baseline (speedup 1.0000x reference)
import functools
import math

import jax
import jax.numpy as jnp
from jax import lax
from jax.experimental import pallas as pl
from jax.experimental.pallas import tpu as pltpu
from jax.experimental.pallas import tpu_sc as plsc

_NUM_CLASSES = 19
_THRESH = -math.log(0.7)
_N_MIN = 131072
_N_PIX = 8 * 512 * 512
_R = 512
_GB = 512 // _R
_GRID = 8 * _GB

_NW = 32
_ROWS_W = 4096 // _NW


def _bincount_body(lbl_hbm, out_hbm, lbl_v, tab_v):
    wid = lax.axis_index("s") * 2 + lax.axis_index("c")
    b = wid // 4
    r0 = (wid % 4) * _ROWS_W
    for c in range(_NUM_CLASSES):
        tab_v[pl.ds(c * 16, 16)] = jnp.zeros((16,), jnp.int32)
    pltpu.sync_copy(lbl_hbm.at[b, pl.ds(r0, _ROWS_W)], lbl_v)
    lane = lax.iota(jnp.int32, 16)
    ones = jnp.ones((16,), jnp.int32)

    def body(i, carry):
        for u in range(32):
            v = lbl_v[i, pl.ds(u * 16, 16)]
            plsc.addupdate_scatter(tab_v, [v * 16 + lane], ones)
        return carry

    lax.fori_loop(0, _ROWS_W, body, 0)
    pltpu.sync_copy(tab_v, out_hbm.at[wid])


@functools.lru_cache(maxsize=None)
def _bincount_call():
    return pl.kernel(
        _bincount_body,
        mesh=plsc.VectorSubcoreMesh(core_axis_name="c", subcore_axis_name="s"),
        out_type=jax.ShapeDtypeStruct((_NW, _NUM_CLASSES * 16), jnp.int32),
        scratch_types=[
            pltpu.VMEM((_ROWS_W, 512), jnp.int32),
            pltpu.VMEM((_NUM_CLASSES * 16,), jnp.int32),
        ],
        compiler_params=pltpu.CompilerParams(needs_layout_passes=False),
    )


def _nll_body(x_ref, lbl_ref, nll_ref):
    x = x_ref[0]
    lbl = lbl_ref[0]
    m = x[0]
    for c in range(1, _NUM_CLASSES):
        m = jnp.maximum(m, x[c])
    s = jnp.exp(x[0] - m)
    for c in range(1, _NUM_CLASSES):
        s = s + jnp.exp(x[c] - m)
    lse = jnp.log(s) + m
    acc_x = x[0]
    for c in range(1, _NUM_CLASSES):
        acc_x = jnp.where(lbl == c, x[c], acc_x)
    nll_ref[0] = lse - acc_x


_nll_call = pl.pallas_call(
    _nll_body,
    grid=(_GRID,),
    in_specs=[
        pl.BlockSpec((1, _NUM_CLASSES, _R, 512),
                     lambda i: (i // _GB, 0, i % _GB, 0)),
        pl.BlockSpec((1, _R, 512), lambda i: (i // _GB, i % _GB, 0)),
    ],
    out_specs=pl.BlockSpec((1, _R, 512), lambda i: (i // _GB, i % _GB, 0)),
    out_shape=jax.ShapeDtypeStruct((8, 512, 512), jnp.float32),
    compiler_params=pltpu.CompilerParams(dimension_semantics=("parallel",)),
)


def _wsel(w_ref, lbl):
    acc_w = jnp.full(lbl.shape, w_ref[0], jnp.float32)
    for c in range(1, _NUM_CLASSES):
        acc_w = jnp.where(lbl == c, w_ref[c], acc_w)
    return acc_w


def _stats_body(w_ref, nll_ref, lbl_ref, sum_ref, cnt_ref):
    i = pl.program_id(0)

    @pl.when(i == 0)
    def _():
        sum_ref[...] = jnp.zeros((8, 128), jnp.float32)
        cnt_ref[...] = jnp.zeros((8, 128), jnp.float32)

    loss = _wsel(w_ref, lbl_ref[0]) * nll_ref[0]
    mask = loss > _THRESH
    sum_ref[...] += jnp.sum(jnp.where(mask, loss, 0.0))
    cnt_ref[...] += jnp.sum(mask.astype(jnp.float32))


_stats_call = pl.pallas_call(
    _stats_body,
    grid=(8,),
    in_specs=[
        pl.BlockSpec(memory_space=pltpu.SMEM),
        pl.BlockSpec((1, 512, 512), lambda i: (i, 0, 0)),
        pl.BlockSpec((1, 512, 512), lambda i: (i, 0, 0)),
    ],
    out_specs=[
        pl.BlockSpec((8, 128), lambda i: (0, 0)),
        pl.BlockSpec((8, 128), lambda i: (0, 0)),
    ],
    out_shape=[
        jax.ShapeDtypeStruct((8, 128), jnp.float32),
        jax.ShapeDtypeStruct((8, 128), jnp.float32),
    ],
    compiler_params=pltpu.CompilerParams(dimension_semantics=("arbitrary",)),
)


def _topk_sum_body(w_ref, nll_ref, lbl_ref, out_ref):
    x = _wsel(w_ref, lbl_ref[...]) * nll_ref[...]
    bits = lax.bitcast_convert_type(x, jnp.int32)
    k = jnp.float32(_N_MIN)

    def body(j, v):
        cand = jnp.bitwise_or(v, jnp.left_shift(jnp.int32(1), 30 - j))
        ge = jnp.sum((bits >= cand).astype(jnp.float32))
        return jnp.where(ge >= k, cand, v)

    v = lax.fori_loop(0, 31, body, jnp.int32(0))
    vval = lax.bitcast_convert_type(v, jnp.float32)
    gt = bits > v
    c_gt = jnp.sum(gt.astype(jnp.float32))
    s_gt = jnp.sum(jnp.where(gt, x, 0.0))
    res = (s_gt + (k - c_gt) * vval) / k
    out_ref[...] = jnp.full((8, 128), res, jnp.float32)


_topk_sum_call = pl.pallas_call(
    _topk_sum_body,
    in_specs=[
        pl.BlockSpec(memory_space=pltpu.SMEM),
        pl.BlockSpec((2048, 1024), lambda: (0, 0)),
        pl.BlockSpec((2048, 1024), lambda: (0, 0)),
    ],
    out_shape=jax.ShapeDtypeStruct((8, 128), jnp.float32),
)


def kernel(logits, labels):
    tabs = _bincount_call()(labels)
    nll = _nll_call(logits, labels)
    counts = jnp.sum(tabs.reshape(_NW, _NUM_CLASSES, 16),
                     axis=(0, 2)).astype(jnp.float32)
    w = (1.0 / jnp.log(1.02 + counts / _N_PIX)).astype(jnp.float32)

    psum, pcnt = _stats_call(w, nll, labels)
    sum_gt = psum[0, 0]
    cnt = pcnt[0, 0]

    def thresh_branch(_):
        return sum_gt / cnt

    def topk_branch(_):
        return _topk_sum_call(w, nll.reshape(2048, 1024),
                              labels.reshape(2048, 1024))[0, 0]

    return lax.cond(cnt > _N_MIN, thresh_branch, topk_branch, None)


# speedup vs baseline: 1.1158x; 1.1158x over previous
import functools
import math

import jax
import jax.numpy as jnp
from jax import lax
from jax.experimental import pallas as pl
from jax.experimental.pallas import tpu as pltpu
from jax.experimental.pallas import tpu_sc as plsc

_NUM_CLASSES = 19
_THRESH = -math.log(0.7)
_N_MIN = 131072
_N_PIX = 8 * 512 * 512
_R = 256
_GB = 512 // _R
_GRID = 8 * _GB

_NW = 32
_ROWS_W = 4096 // _NW


def _bincount_body(lbl_hbm, out_hbm, lbl_v, tab_v):
    wid = lax.axis_index("s") * 2 + lax.axis_index("c")
    b = wid // 4
    r0 = (wid % 4) * _ROWS_W
    for c in range(_NUM_CLASSES):
        tab_v[pl.ds(c * 16, 16)] = jnp.zeros((16,), jnp.int32)
    pltpu.sync_copy(lbl_hbm.at[b, pl.ds(r0, _ROWS_W)], lbl_v)
    lane = lax.iota(jnp.int32, 16)
    ones = jnp.ones((16,), jnp.int32)

    def body(i, carry):
        for u in range(32):
            v = lbl_v[i, pl.ds(u * 16, 16)]
            plsc.addupdate_scatter(tab_v, [v * 16 + lane], ones)
        return carry

    lax.fori_loop(0, _ROWS_W, body, 0)
    pltpu.sync_copy(tab_v, out_hbm.at[wid])


@functools.lru_cache(maxsize=None)
def _bincount_call():
    return pl.kernel(
        _bincount_body,
        mesh=plsc.VectorSubcoreMesh(core_axis_name="c", subcore_axis_name="s"),
        out_type=jax.ShapeDtypeStruct((_NW, _NUM_CLASSES * 16), jnp.int32),
        scratch_types=[
            pltpu.VMEM((_ROWS_W, 512), jnp.int32),
            pltpu.VMEM((_NUM_CLASSES * 16,), jnp.int32),
        ],
        compiler_params=pltpu.CompilerParams(needs_layout_passes=False),
    )


def _nll_body(x_ref, lbl_ref, nll_ref):
    x = x_ref[0]
    lbl = lbl_ref[0]
    m = x[0]
    for c in range(1, _NUM_CLASSES):
        m = jnp.maximum(m, x[c])
    s = jnp.exp(x[0] - m)
    for c in range(1, _NUM_CLASSES):
        s = s + jnp.exp(x[c] - m)
    lse = jnp.log(s) + m
    acc_x = x[0]
    for c in range(1, _NUM_CLASSES):
        acc_x = jnp.where(lbl == c, x[c], acc_x)
    nll_ref[0] = lse - acc_x


_nll_call = pl.pallas_call(
    _nll_body,
    grid=(_GRID,),
    in_specs=[
        pl.BlockSpec((1, _NUM_CLASSES, _R, 512),
                     lambda i: (i // _GB, 0, i % _GB, 0)),
        pl.BlockSpec((1, _R, 512), lambda i: (i // _GB, i % _GB, 0)),
    ],
    out_specs=pl.BlockSpec((1, _R, 512), lambda i: (i // _GB, i % _GB, 0)),
    out_shape=jax.ShapeDtypeStruct((8, 512, 512), jnp.float32),
    compiler_params=pltpu.CompilerParams(dimension_semantics=("parallel",)),
)


def _wsel(w_ref, lbl):
    wt = jnp.broadcast_to(w_ref[0, :], (lbl.shape[0], _NUM_CLASSES))
    return jnp.take_along_axis(wt, lbl, axis=1)


def _stats_body(w_ref, nll_ref, lbl_ref, sum_ref, cnt_ref):
    i = pl.program_id(0)

    @pl.when(i == 0)
    def _():
        sum_ref[...] = jnp.zeros((8, 128), jnp.float32)
        cnt_ref[...] = jnp.zeros((8, 128), jnp.float32)

    loss = _wsel(w_ref, lbl_ref[0]) * nll_ref[0]
    mask = loss > _THRESH
    sum_ref[...] += jnp.sum(jnp.where(mask, loss, 0.0))
    cnt_ref[...] += jnp.sum(mask.astype(jnp.float32))


_stats_call = pl.pallas_call(
    _stats_body,
    grid=(8,),
    in_specs=[
        pl.BlockSpec((1, _NUM_CLASSES), lambda i: (0, 0)),
        pl.BlockSpec((1, 512, 512), lambda i: (i, 0, 0)),
        pl.BlockSpec((1, 512, 512), lambda i: (i, 0, 0)),
    ],
    out_specs=[
        pl.BlockSpec((8, 128), lambda i: (0, 0)),
        pl.BlockSpec((8, 128), lambda i: (0, 0)),
    ],
    out_shape=[
        jax.ShapeDtypeStruct((8, 128), jnp.float32),
        jax.ShapeDtypeStruct((8, 128), jnp.float32),
    ],
    compiler_params=pltpu.CompilerParams(dimension_semantics=("arbitrary",)),
)


def _topk_sum_body(w_ref, nll_ref, lbl_ref, out_ref):
    x = _wsel(w_ref, lbl_ref[...]) * nll_ref[...]
    bits = lax.bitcast_convert_type(x, jnp.int32)
    k = jnp.float32(_N_MIN)

    def body(j, v):
        cand = jnp.bitwise_or(v, jnp.left_shift(jnp.int32(1), 30 - j))
        ge = jnp.sum((bits >= cand).astype(jnp.float32))
        return jnp.where(ge >= k, cand, v)

    v = lax.fori_loop(0, 31, body, jnp.int32(0))
    vval = lax.bitcast_convert_type(v, jnp.float32)
    gt = bits > v
    c_gt = jnp.sum(gt.astype(jnp.float32))
    s_gt = jnp.sum(jnp.where(gt, x, 0.0))
    res = (s_gt + (k - c_gt) * vval) / k
    out_ref[...] = jnp.full((8, 128), res, jnp.float32)


_topk_sum_call = pl.pallas_call(
    _topk_sum_body,
    in_specs=[
        pl.BlockSpec((1, _NUM_CLASSES), lambda: (0, 0)),
        pl.BlockSpec((2048, 1024), lambda: (0, 0)),
        pl.BlockSpec((2048, 1024), lambda: (0, 0)),
    ],
    out_shape=jax.ShapeDtypeStruct((8, 128), jnp.float32),
)


def kernel(logits, labels):
    tabs = _bincount_call()(labels)
    nll = _nll_call(logits, labels)
    counts = jnp.sum(tabs.reshape(_NW, _NUM_CLASSES, 16),
                     axis=(0, 2)).astype(jnp.float32)
    w = (1.0 / jnp.log(1.02 + counts / _N_PIX)).astype(jnp.float32)
    w = w.reshape(1, _NUM_CLASSES)

    psum, pcnt = _stats_call(w, nll, labels)
    sum_gt = psum[0, 0]
    cnt = pcnt[0, 0]

    def thresh_branch(_):
        return sum_gt / cnt

    def topk_branch(_):
        return _topk_sum_call(w, nll.reshape(2048, 1024),
                              labels.reshape(2048, 1024))[0, 0]

    return lax.cond(cnt > _N_MIN, thresh_branch, topk_branch, None)


# speedup vs baseline: 1.1456x; 1.0267x over previous
import functools
import math

import jax
import jax.numpy as jnp
from jax import lax
from jax.experimental import pallas as pl
from jax.experimental.pallas import tpu as pltpu
from jax.experimental.pallas import tpu_sc as plsc

_NUM_CLASSES = 19
_THRESH = -math.log(0.7)
_N_MIN = 131072
_N_PIX = 8 * 512 * 512
_R = 256
_GB = 512 // _R
_GRID = 8 * _GB

_NW = 32
_ROWS_W = 4096 // _NW


def _bincount_body(lbl_hbm, out_hbm, lbl_v, tab_v, cnt_v):
    wid = lax.axis_index("s") * 2 + lax.axis_index("c")
    b = wid // 4
    r0 = (wid % 4) * _ROWS_W
    for c in range(32):
        tab_v[pl.ds(c * 16, 16)] = jnp.zeros((16,), jnp.int32)
    pltpu.sync_copy(lbl_hbm.at[b, pl.ds(r0, _ROWS_W)], lbl_v)
    lane = lax.iota(jnp.int32, 16)
    ones = jnp.ones((16,), jnp.int32)

    def body(i, carry):
        for u in range(32):
            v = lbl_v[i, pl.ds(u * 16, 16)]
            plsc.addupdate_scatter(tab_v, [v * 16 + lane], ones)
        return carry

    lax.fori_loop(0, _ROWS_W, body, 0)
    for g in range(2):
        base = lane * 16 + g * 256
        acc = jnp.zeros((16,), jnp.int32)
        for j in range(16):
            acc = acc + plsc.load_gather(tab_v, [base + j])
        cnt_v[pl.ds(g * 16, 16)] = acc
    pltpu.sync_copy(cnt_v, out_hbm.at[wid])


@functools.lru_cache(maxsize=None)
def _bincount_call():
    return pl.kernel(
        _bincount_body,
        mesh=plsc.VectorSubcoreMesh(core_axis_name="c", subcore_axis_name="s"),
        out_type=jax.ShapeDtypeStruct((_NW, 32), jnp.int32),
        scratch_types=[
            pltpu.VMEM((_ROWS_W, 512), jnp.int32),
            pltpu.VMEM((512,), jnp.int32),
            pltpu.VMEM((32,), jnp.int32),
        ],
        compiler_params=pltpu.CompilerParams(needs_layout_passes=False),
    )


def _nll_body(x_ref, lbl_ref, nll_ref):
    x = x_ref[0]
    lbl = lbl_ref[0]
    m = x[0]
    for c in range(1, _NUM_CLASSES):
        m = jnp.maximum(m, x[c])
    s = jnp.exp(x[0] - m)
    for c in range(1, _NUM_CLASSES):
        s = s + jnp.exp(x[c] - m)
    lse = jnp.log(s) + m
    acc_x = x[0]
    for c in range(1, _NUM_CLASSES):
        acc_x = jnp.where(lbl == c, x[c], acc_x)
    nll_ref[0] = lse - acc_x


_nll_call = pl.pallas_call(
    _nll_body,
    grid=(_GRID,),
    in_specs=[
        pl.BlockSpec((1, _NUM_CLASSES, _R, 512),
                     lambda i: (i // _GB, 0, i % _GB, 0)),
        pl.BlockSpec((1, _R, 512), lambda i: (i // _GB, i % _GB, 0)),
    ],
    out_specs=pl.BlockSpec((1, _R, 512), lambda i: (i // _GB, i % _GB, 0)),
    out_shape=jax.ShapeDtypeStruct((8, 512, 512), jnp.float32),
    compiler_params=pltpu.CompilerParams(dimension_semantics=("parallel",)),
)


def _weights_from_tabs(tabs_ref):
    s = jnp.sum(tabs_ref[...].astype(jnp.float32), axis=0)
    counts = s[:_NUM_CLASSES]
    return 1.0 / jnp.log(1.02 + counts * (1.0 / _N_PIX))


def _wsel(w19, lbl):
    wt = jnp.broadcast_to(w19, (lbl.shape[0], _NUM_CLASSES))
    return jnp.take_along_axis(wt, lbl, axis=1)


def _stats_body(tabs_ref, nll_ref, lbl_ref, sum_ref, cnt_ref, res_ref):
    i = pl.program_id(0)

    @pl.when(i == 0)
    def _():
        sum_ref[...] = jnp.zeros((8, 128), jnp.float32)
        cnt_ref[...] = jnp.zeros((8, 128), jnp.float32)

    w19 = _weights_from_tabs(tabs_ref)
    loss = _wsel(w19, lbl_ref[0]) * nll_ref[0]
    mask = loss > _THRESH
    sum_ref[...] += jnp.sum(jnp.where(mask, loss, 0.0))
    cnt_ref[...] += jnp.sum(mask.astype(jnp.float32))

    @pl.when(i == 7)
    def _():
        res_ref[...] = sum_ref[...] / cnt_ref[...]


_stats_call = pl.pallas_call(
    _stats_body,
    grid=(8,),
    in_specs=[
        pl.BlockSpec((_NW, 32), lambda i: (0, 0)),
        pl.BlockSpec((1, 512, 512), lambda i: (i, 0, 0)),
        pl.BlockSpec((1, 512, 512), lambda i: (i, 0, 0)),
    ],
    out_specs=[
        pl.BlockSpec((8, 128), lambda i: (0, 0)),
        pl.BlockSpec((8, 128), lambda i: (0, 0)),
        pl.BlockSpec((8, 128), lambda i: (0, 0)),
    ],
    out_shape=[
        jax.ShapeDtypeStruct((8, 128), jnp.float32),
        jax.ShapeDtypeStruct((8, 128), jnp.float32),
        jax.ShapeDtypeStruct((8, 128), jnp.float32),
    ],
    compiler_params=pltpu.CompilerParams(dimension_semantics=("arbitrary",)),
)


def _topk_sum_body(tabs_ref, nll_ref, lbl_ref, out_ref):
    w19 = _weights_from_tabs(tabs_ref)
    x = _wsel(w19, lbl_ref[...]) * nll_ref[...]
    bits = lax.bitcast_convert_type(x, jnp.int32)
    k = jnp.float32(_N_MIN)

    def body(j, v):
        cand = jnp.bitwise_or(v, jnp.left_shift(jnp.int32(1), 30 - j))
        ge = jnp.sum((bits >= cand).astype(jnp.float32))
        return jnp.where(ge >= k, cand, v)

    v = lax.fori_loop(0, 31, body, jnp.int32(0))
    vval = lax.bitcast_convert_type(v, jnp.float32)
    gt = bits > v
    c_gt = jnp.sum(gt.astype(jnp.float32))
    s_gt = jnp.sum(jnp.where(gt, x, 0.0))
    res = (s_gt + (k - c_gt) * vval) / k
    out_ref[...] = jnp.full((8, 128), res, jnp.float32)


_topk_sum_call = pl.pallas_call(
    _topk_sum_body,
    in_specs=[
        pl.BlockSpec((_NW, 32), lambda: (0, 0)),
        pl.BlockSpec((2048, 1024), lambda: (0, 0)),
        pl.BlockSpec((2048, 1024), lambda: (0, 0)),
    ],
    out_shape=jax.ShapeDtypeStruct((8, 128), jnp.float32),
)


def kernel(logits, labels):
    tabs = _bincount_call()(labels)
    nll = _nll_call(logits, labels)

    psum, pcnt, res = _stats_call(tabs, nll, labels)
    cnt = pcnt[0, 0]

    def thresh_branch(_):
        return res[0, 0]

    def topk_branch(_):
        return _topk_sum_call(tabs, nll.reshape(2048, 1024),
                              labels.reshape(2048, 1024))[0, 0]

    return lax.cond(cnt > _N_MIN, thresh_branch, topk_branch, None)


# speedup vs baseline: 1.1621x; 1.0144x over previous
import functools
import math

import jax
import jax.numpy as jnp
from jax import lax
from jax.experimental import pallas as pl
from jax.experimental.pallas import tpu as pltpu
from jax.experimental.pallas import tpu_sc as plsc

_NUM_CLASSES = 19
_THRESH = -math.log(0.7)
_N_MIN = 131072
_N_PIX = 8 * 512 * 512
_R = 256
_GB = 512 // _R
_GRID = 8 * _GB

_NW = 32
_ROWS_W = 4096 // _NW


def _bincount_body(lbl_hbm, out_hbm, lbl_v, tab_v, cnt_v):
    wid = lax.axis_index("s") * 2 + lax.axis_index("c")
    b = wid // 4
    r0 = (wid % 4) * _ROWS_W
    for c in range(32):
        tab_v[pl.ds(c * 16, 16)] = jnp.zeros((16,), jnp.int32)
    pltpu.sync_copy(lbl_hbm.at[b, pl.ds(r0, _ROWS_W)], lbl_v)
    lane = lax.iota(jnp.int32, 16)
    ones = jnp.ones((16,), jnp.int32)

    def body(i, carry):
        for u in range(32):
            v = lbl_v[i, pl.ds(u * 16, 16)]
            plsc.addupdate_scatter(tab_v, [v * 16 + lane], ones)
        return carry

    lax.fori_loop(0, _ROWS_W, body, 0)
    for g in range(2):
        base = lane * 16 + g * 256
        acc = jnp.zeros((16,), jnp.int32)
        for j in range(16):
            acc = acc + plsc.load_gather(tab_v, [base + j])
        cnt_v[pl.ds(g * 16, 16)] = acc
    pltpu.sync_copy(cnt_v, out_hbm.at[wid])


@functools.lru_cache(maxsize=None)
def _bincount_call():
    return pl.kernel(
        _bincount_body,
        mesh=plsc.VectorSubcoreMesh(core_axis_name="c", subcore_axis_name="s"),
        out_type=jax.ShapeDtypeStruct((_NW, 32), jnp.int32),
        scratch_types=[
            pltpu.VMEM((_ROWS_W, 512), jnp.int32),
            pltpu.VMEM((512,), jnp.int32),
            pltpu.VMEM((32,), jnp.int32),
        ],
        compiler_params=pltpu.CompilerParams(needs_layout_passes=False),
    )


def _nll_body(x_ref, lbl_ref, nll_ref):
    x = x_ref[0]
    lbl = lbl_ref[0]
    m = x[0]
    for c in range(1, _NUM_CLASSES):
        m = jnp.maximum(m, x[c])
    s = jnp.exp(x[0] - m)
    for c in range(1, _NUM_CLASSES):
        s = s + jnp.exp(x[c] - m)
    lse = jnp.log(s) + m
    acc_x = x[0]
    for c in range(1, _NUM_CLASSES):
        acc_x = jnp.where(lbl == c, x[c], acc_x)
    nll_ref[0] = lse - acc_x


_nll_call = pl.pallas_call(
    _nll_body,
    grid=(_GRID,),
    in_specs=[
        pl.BlockSpec((1, _NUM_CLASSES, _R, 512),
                     lambda i: (i // _GB, 0, i % _GB, 0)),
        pl.BlockSpec((1, _R, 512), lambda i: (i // _GB, i % _GB, 0)),
    ],
    out_specs=pl.BlockSpec((1, _R, 512), lambda i: (i // _GB, i % _GB, 0)),
    out_shape=jax.ShapeDtypeStruct((8, 512, 512), jnp.float32),
    compiler_params=pltpu.CompilerParams(dimension_semantics=("parallel",)),
)


def _weights_from_tabs(tabs_ref):
    s = jnp.sum(tabs_ref[...].astype(jnp.float32), axis=0)
    counts = s[:_NUM_CLASSES]
    return 1.0 / jnp.log(1.02 + counts * (1.0 / _N_PIX))


def _wsel(w19, lbl):
    wt = jnp.broadcast_to(w19, lbl.shape[:-1] + (_NUM_CLASSES,))
    return jnp.take_along_axis(wt, lbl, axis=-1)


def _stats_body(tabs_ref, nll_ref, lbl_ref, sum_ref, cnt_ref, res_ref):
    i = pl.program_id(0)

    @pl.when(i == 0)
    def _():
        sum_ref[...] = jnp.zeros((8, 128), jnp.float32)
        cnt_ref[...] = jnp.zeros((8, 128), jnp.float32)

    w19 = _weights_from_tabs(tabs_ref)
    loss = _wsel(w19, lbl_ref[...]) * nll_ref[...]
    mask = loss > _THRESH
    sum_ref[...] += jnp.sum(jnp.where(mask, loss, 0.0))
    cnt_ref[...] += jnp.sum(mask.astype(jnp.float32))

    @pl.when(i == pl.num_programs(0) - 1)
    def _():
        res_ref[...] = sum_ref[...] / cnt_ref[...]


_stats_call = pl.pallas_call(
    _stats_body,
    grid=(4,),
    in_specs=[
        pl.BlockSpec((_NW, 32), lambda i: (0, 0)),
        pl.BlockSpec((2, 512, 512), lambda i: (i, 0, 0)),
        pl.BlockSpec((2, 512, 512), lambda i: (i, 0, 0)),
    ],
    out_specs=[
        pl.BlockSpec((8, 128), lambda i: (0, 0)),
        pl.BlockSpec((8, 128), lambda i: (0, 0)),
        pl.BlockSpec((8, 128), lambda i: (0, 0)),
    ],
    out_shape=[
        jax.ShapeDtypeStruct((8, 128), jnp.float32),
        jax.ShapeDtypeStruct((8, 128), jnp.float32),
        jax.ShapeDtypeStruct((8, 128), jnp.float32),
    ],
    compiler_params=pltpu.CompilerParams(dimension_semantics=("arbitrary",)),
)


def _topk_sum_body(tabs_ref, nll_ref, lbl_ref, out_ref):
    w19 = _weights_from_tabs(tabs_ref)
    x = _wsel(w19, lbl_ref[...]) * nll_ref[...]
    bits = lax.bitcast_convert_type(x, jnp.int32)
    k = jnp.float32(_N_MIN)

    def body(j, v):
        cand = jnp.bitwise_or(v, jnp.left_shift(jnp.int32(1), 30 - j))
        ge = jnp.sum((bits >= cand).astype(jnp.float32))
        return jnp.where(ge >= k, cand, v)

    v = lax.fori_loop(0, 31, body, jnp.int32(0))
    vval = lax.bitcast_convert_type(v, jnp.float32)
    gt = bits > v
    c_gt = jnp.sum(gt.astype(jnp.float32))
    s_gt = jnp.sum(jnp.where(gt, x, 0.0))
    res = (s_gt + (k - c_gt) * vval) / k
    out_ref[...] = jnp.full((8, 128), res, jnp.float32)


_topk_sum_call = pl.pallas_call(
    _topk_sum_body,
    in_specs=[
        pl.BlockSpec((_NW, 32), lambda: (0, 0)),
        pl.BlockSpec((2048, 1024), lambda: (0, 0)),
        pl.BlockSpec((2048, 1024), lambda: (0, 0)),
    ],
    out_shape=jax.ShapeDtypeStruct((8, 128), jnp.float32),
)


def kernel(logits, labels):
    tabs = _bincount_call()(labels)
    nll = _nll_call(logits, labels)

    psum, pcnt, res = _stats_call(tabs, nll, labels)
    cnt = pcnt[0, 0]

    def thresh_branch(_):
        return res[0, 0]

    def topk_branch(_):
        return _topk_sum_call(tabs, nll.reshape(2048, 1024),
                              labels.reshape(2048, 1024))[0, 0]

    return lax.cond(cnt > _N_MIN, thresh_branch, topk_branch, None)
